# baseline (device time: 39217 ns/iter reference)
import jax
import jax.numpy as jnp
from jax import lax
from jax.experimental import pallas as pl
from jax.experimental.pallas import tpu as pltpu

N_DEV = 8
SQ = 256
SKV = 4096
D_MODEL = 1024
DH = 128
H_PER = 8
CHW = D_MODEL // N_DEV
SCALE = 0.08838834764831843


def kernel(x, Wq, Wo, K_ext, V_ext):
    def body(x_ref, wq_ref, wo_ref, k_hbm, v_hbm, out_ref,
             k_vmem, v_vmem, send_buf, rs_buf, ag_buf,
             copy_sems, send_sems, rs_sems, ag_sems):
        my = lax.axis_index("i")
        h0 = my * H_PER

        def make_cps(h):
            slot = h % 2
            kcp = pltpu.make_async_copy(
                k_hbm.at[0, :, h0 + h, :], k_vmem.at[slot],
                copy_sems.at[slot])
            vcp = pltpu.make_async_copy(
                v_hbm.at[0, :, h0 + h, :], v_vmem.at[slot],
                copy_sems.at[2 + slot])
            return kcp, vcp

        cps = [make_cps(0)]
        cps[0][0].start()
        cps[0][1].start()

        barrier = pltpu.get_barrier_semaphore()
        for k in range(1, N_DEV):
            pl.semaphore_signal(barrier, inc=1,
                                device_id=((my + k) % N_DEV,),
                                device_id_type=pl.DeviceIdType.MESH)
        pl.semaphore_wait(barrier, N_DEV - 1)

        q = jnp.dot(x_ref[0].astype(jnp.bfloat16),
                    wq_ref[...].astype(jnp.bfloat16),
                    preferred_element_type=jnp.float32)

        outs = []
        for h in range(H_PER):
            if h + 1 < H_PER:
                nxt = make_cps(h + 1)
                nxt[0].start()
                nxt[1].start()
                cps.append(nxt)
            cps[h][0].wait()
            cps[h][1].wait()
            slot = h % 2
            qh = q[:, h * DH:(h + 1) * DH].astype(jnp.bfloat16)
            s = lax.dot_general(
                qh, k_vmem[slot].astype(jnp.bfloat16),
                (((1,), (1,)), ((), ())),
                preferred_element_type=jnp.float32) * SCALE
            p = jnp.exp(s)
            l = jnp.sum(p, axis=-1, keepdims=True)
            o = jnp.dot(p.astype(jnp.bfloat16),
                        v_vmem[slot].astype(jnp.bfloat16),
                        preferred_element_type=jnp.float32) / l
            outs.append(o)
        attn = jnp.concatenate(outs, axis=1).astype(jnp.bfloat16)

        sends = []

        for k in range(1, N_DEV):
            p = (my + k) % N_DEV
            chunk = jnp.dot(attn, wo_ref[:, pl.ds(p * CHW, CHW)].astype(
                jnp.bfloat16), preferred_element_type=jnp.float32)
            send_buf[k - 1] = chunk.astype(jnp.bfloat16)
            rs = pltpu.make_async_remote_copy(
                src_ref=send_buf.at[k - 1],
                dst_ref=rs_buf.at[my],
                send_sem=send_sems.at[k - 1],
                recv_sem=rs_sems.at[my],
                device_id=(p,),
                device_id_type=pl.DeviceIdType.MESH,
            )
            rs.start()
            sends.append(rs)

        own = jnp.dot(attn, wo_ref[:, pl.ds(my * CHW, CHW)].astype(
            jnp.bfloat16), preferred_element_type=jnp.float32)
        rs_buf[my] = jnp.zeros((SQ, CHW), jnp.bfloat16)

        for k in range(1, N_DEV):
            s = (my + k) % N_DEV
            pltpu.make_async_remote_copy(
                src_ref=rs_buf.at[s],
                dst_ref=rs_buf.at[s],
                send_sem=send_sems.at[k - 1],
                recv_sem=rs_sems.at[s],
                device_id=(s,),
                device_id_type=pl.DeviceIdType.MESH,
            ).wait_recv()

        red = own + jnp.sum(rs_buf[...].astype(jnp.float32), axis=0)

        ag_buf[:, pl.ds(my * CHW, CHW)] = red.astype(jnp.bfloat16)
        for k in range(1, N_DEV):
            p = (my + k) % N_DEV
            ag = pltpu.make_async_remote_copy(
                src_ref=ag_buf.at[:, pl.ds(my * CHW, CHW)],
                dst_ref=ag_buf.at[:, pl.ds(my * CHW, CHW)],
                send_sem=send_sems.at[N_DEV - 1 + k - 1],
                recv_sem=ag_sems.at[my],
                device_id=(p,),
                device_id_type=pl.DeviceIdType.MESH,
            )
            ag.start()
            sends.append(ag)

        for k in range(1, N_DEV):
            s = (my + k) % N_DEV
            pltpu.make_async_remote_copy(
                src_ref=ag_buf.at[:, pl.ds(s * CHW, CHW)],
                dst_ref=ag_buf.at[:, pl.ds(s * CHW, CHW)],
                send_sem=send_sems.at[k - 1],
                recv_sem=ag_sems.at[s],
                device_id=(s,),
                device_id_type=pl.DeviceIdType.MESH,
            ).wait_recv()

        out_ref[0] = ag_buf[...].astype(jnp.float32)
        out_ref[0, :, pl.ds(my * CHW, CHW)] = red

        for rdma in sends:
            rdma.wait_send()

    return pl.pallas_call(
        body,
        out_shape=jax.ShapeDtypeStruct((1, SQ, D_MODEL), jnp.float32),
        in_specs=[
            pl.BlockSpec(memory_space=pltpu.VMEM),
            pl.BlockSpec(memory_space=pltpu.VMEM),
            pl.BlockSpec(memory_space=pltpu.VMEM),
            pl.BlockSpec(memory_space=pl.ANY),
            pl.BlockSpec(memory_space=pl.ANY),
        ],
        out_specs=pl.BlockSpec(memory_space=pltpu.VMEM),
        scratch_shapes=[
            pltpu.VMEM((2, SKV, DH), jnp.float32),
            pltpu.VMEM((2, SKV, DH), jnp.float32),
            pltpu.VMEM((N_DEV - 1, SQ, CHW), jnp.bfloat16),
            pltpu.VMEM((N_DEV, SQ, CHW), jnp.bfloat16),
            pltpu.VMEM((SQ, D_MODEL), jnp.bfloat16),
            pltpu.SemaphoreType.DMA((4,)),
            pltpu.SemaphoreType.DMA((2 * (N_DEV - 1),)),
            pltpu.SemaphoreType.DMA((N_DEV,)),
            pltpu.SemaphoreType.DMA((N_DEV,)),
        ],
        compiler_params=pltpu.CompilerParams(
            collective_id=0,
            vmem_limit_bytes=100 * 1024 * 1024,
        ),
    )(x, Wq, Wo, K_ext, V_ext)


# device time: 38382 ns/iter; 1.0218x vs baseline; 1.0218x over previous
import jax
import jax.numpy as jnp
from jax import lax
from jax.experimental import pallas as pl
from jax.experimental.pallas import tpu as pltpu

N_DEV = 8
SQ = 256
SKV = 4096
D_MODEL = 1024
DH = 128
H_PER = 8
CHW = D_MODEL // N_DEV
SCALE = 0.08838834764831843
LOG2E = 1.4426950408889634


def kernel(x, Wq, Wo, K_ext, V_ext):
    def body(x_ref, wq_ref, wo_ref, k_hbm, v_hbm, out_ref,
             k_vmem, v_vmem, send_buf, rs_buf, ag_buf,
             copy_sems, send_sems, rs_sems, ag_sems):
        my = lax.axis_index("i")
        h0 = my * H_PER

        def make_cps(h):
            slot = h % 2
            kcp = pltpu.make_async_copy(
                k_hbm.at[0, :, h0 + h, :], k_vmem.at[slot],
                copy_sems.at[slot])
            vcp = pltpu.make_async_copy(
                v_hbm.at[0, :, h0 + h, :], v_vmem.at[slot],
                copy_sems.at[2 + slot])
            return kcp, vcp

        cps = [make_cps(0)]
        cps[0][0].start()
        cps[0][1].start()

        barrier = pltpu.get_barrier_semaphore()
        for k in range(1, N_DEV):
            pl.semaphore_signal(barrier, inc=1,
                                device_id=((my + k) % N_DEV,),
                                device_id_type=pl.DeviceIdType.MESH)
        pl.semaphore_wait(barrier, N_DEV - 1)

        q = (jnp.dot(x_ref[0].astype(jnp.bfloat16),
                     wq_ref[...].astype(jnp.bfloat16),
                     preferred_element_type=jnp.float32)
             * (SCALE * LOG2E)).astype(jnp.bfloat16)

        outs = []
        for h in range(H_PER):
            if h + 1 < H_PER:
                nxt = make_cps(h + 1)
                nxt[0].start()
                nxt[1].start()
                cps.append(nxt)
            cps[h][0].wait()
            cps[h][1].wait()
            slot = h % 2
            qh = q[:, h * DH:(h + 1) * DH]
            s = lax.dot_general(
                qh, k_vmem[slot].astype(jnp.bfloat16),
                (((1,), (1,)), ((), ())),
                preferred_element_type=jnp.float32)
            p = jnp.exp2(s)
            l = jnp.sum(p, axis=-1, keepdims=True)
            o = jnp.dot(p.astype(jnp.bfloat16),
                        v_vmem[slot].astype(jnp.bfloat16),
                        preferred_element_type=jnp.float32) / l
            outs.append(o)
        attn = jnp.concatenate(outs, axis=1).astype(jnp.bfloat16)

        sends = []

        for k in range(1, N_DEV):
            p = (my + k) % N_DEV
            chunk = jnp.dot(attn, wo_ref[:, pl.ds(p * CHW, CHW)].astype(
                jnp.bfloat16), preferred_element_type=jnp.float32)
            send_buf[k - 1] = chunk.astype(jnp.bfloat16)
            rs = pltpu.make_async_remote_copy(
                src_ref=send_buf.at[k - 1],
                dst_ref=rs_buf.at[my],
                send_sem=send_sems.at[k - 1],
                recv_sem=rs_sems.at[my],
                device_id=(p,),
                device_id_type=pl.DeviceIdType.MESH,
            )
            rs.start()
            sends.append(rs)

        own = jnp.dot(attn, wo_ref[:, pl.ds(my * CHW, CHW)].astype(
            jnp.bfloat16), preferred_element_type=jnp.float32)
        rs_buf[my] = jnp.zeros((SQ, CHW), jnp.bfloat16)

        for k in range(1, N_DEV):
            s = (my + k) % N_DEV
            pltpu.make_async_remote_copy(
                src_ref=rs_buf.at[s],
                dst_ref=rs_buf.at[s],
                send_sem=send_sems.at[k - 1],
                recv_sem=rs_sems.at[s],
                device_id=(s,),
                device_id_type=pl.DeviceIdType.MESH,
            ).wait_recv()

        red = own + jnp.sum(rs_buf[...].astype(jnp.float32), axis=0)

        ag_buf[:, pl.ds(my * CHW, CHW)] = red.astype(jnp.bfloat16)
        for k in range(1, N_DEV):
            p = (my + k) % N_DEV
            ag = pltpu.make_async_remote_copy(
                src_ref=ag_buf.at[:, pl.ds(my * CHW, CHW)],
                dst_ref=ag_buf.at[:, pl.ds(my * CHW, CHW)],
                send_sem=send_sems.at[N_DEV - 1 + k - 1],
                recv_sem=ag_sems.at[my],
                device_id=(p,),
                device_id_type=pl.DeviceIdType.MESH,
            )
            ag.start()
            sends.append(ag)

        out_ref[0, :, pl.ds(my * CHW, CHW)] = red
        for k in range(1, N_DEV):
            s = (my + k) % N_DEV
            pltpu.make_async_remote_copy(
                src_ref=ag_buf.at[:, pl.ds(s * CHW, CHW)],
                dst_ref=ag_buf.at[:, pl.ds(s * CHW, CHW)],
                send_sem=send_sems.at[k - 1],
                recv_sem=ag_sems.at[s],
                device_id=(s,),
                device_id_type=pl.DeviceIdType.MESH,
            ).wait_recv()
            out_ref[0, :, pl.ds(s * CHW, CHW)] = (
                ag_buf[:, pl.ds(s * CHW, CHW)].astype(jnp.float32))

        for rdma in sends:
            rdma.wait_send()

    return pl.pallas_call(
        body,
        out_shape=jax.ShapeDtypeStruct((1, SQ, D_MODEL), jnp.float32),
        in_specs=[
            pl.BlockSpec(memory_space=pltpu.VMEM),
            pl.BlockSpec(memory_space=pltpu.VMEM),
            pl.BlockSpec(memory_space=pltpu.VMEM),
            pl.BlockSpec(memory_space=pl.ANY),
            pl.BlockSpec(memory_space=pl.ANY),
        ],
        out_specs=pl.BlockSpec(memory_space=pltpu.VMEM),
        scratch_shapes=[
            pltpu.VMEM((2, SKV, DH), jnp.float32),
            pltpu.VMEM((2, SKV, DH), jnp.float32),
            pltpu.VMEM((N_DEV - 1, SQ, CHW), jnp.bfloat16),
            pltpu.VMEM((N_DEV, SQ, CHW), jnp.bfloat16),
            pltpu.VMEM((SQ, D_MODEL), jnp.bfloat16),
            pltpu.SemaphoreType.DMA((4,)),
            pltpu.SemaphoreType.DMA((2 * (N_DEV - 1),)),
            pltpu.SemaphoreType.DMA((N_DEV,)),
            pltpu.SemaphoreType.DMA((N_DEV,)),
        ],
        compiler_params=pltpu.CompilerParams(
            collective_id=0,
            vmem_limit_bytes=100 * 1024 * 1024,
        ),
    )(x, Wq, Wo, K_ext, V_ext)


# device time: 38092 ns/iter; 1.0295x vs baseline; 1.0076x over previous
import jax
import jax.numpy as jnp
from jax import lax
from jax.experimental import pallas as pl
from jax.experimental.pallas import tpu as pltpu

N_DEV = 8
SQ = 256
SKV = 4096
D_MODEL = 1024
DH = 128
H_PER = 8
CHW = D_MODEL // N_DEV
ROWB = 2
RB = SQ // ROWB
SCALE = 0.08838834764831843
LOG2E = 1.4426950408889634


def kernel(x, Wq, Wo, K_ext, V_ext):
    def body(x_ref, wq_ref, wo_ref, k_hbm, v_hbm, out_ref,
             k_vmem, v_vmem, send_buf, rs_buf, ag_buf,
             copy_sems, send_sems, rs_sems, ag_sems):
        my = lax.axis_index("i")
        h0 = my * H_PER

        kcps = []
        vcps = []
        for h in range(H_PER):
            kcp = pltpu.make_async_copy(
                k_hbm.at[0, :, h0 + h, :], k_vmem.at[h], copy_sems.at[h])
            vcp = pltpu.make_async_copy(
                v_hbm.at[0, :, h0 + h, :], v_vmem.at[h],
                copy_sems.at[H_PER + h])
            kcp.start()
            vcp.start()
            kcps.append(kcp)
            vcps.append(vcp)

        barrier = pltpu.get_barrier_semaphore()
        for k in range(1, N_DEV):
            pl.semaphore_signal(barrier, inc=1,
                                device_id=((my + k) % N_DEV,),
                                device_id_type=pl.DeviceIdType.MESH)
        pl.semaphore_wait(barrier, N_DEV - 1)

        q = (jnp.dot(x_ref[0].astype(jnp.bfloat16),
                     wq_ref[...].astype(jnp.bfloat16),
                     preferred_element_type=jnp.float32)
             * (SCALE * LOG2E)).astype(jnp.bfloat16)

        sends = []
        own = [None] * ROWB

        for rb in range(ROWB):
            r0 = rb * RB
            outs = []
            for h in range(H_PER):
                if rb == 0:
                    kcps[h].wait()
                    vcps[h].wait()
                qh = q[r0:r0 + RB, h * DH:(h + 1) * DH]
                s = lax.dot_general(
                    qh, k_vmem[h].astype(jnp.bfloat16),
                    (((1,), (1,)), ((), ())),
                    preferred_element_type=jnp.float32)
                p = jnp.exp2(s)
                l = jnp.sum(p, axis=-1, keepdims=True)
                o = jnp.dot(p.astype(jnp.bfloat16),
                            v_vmem[h].astype(jnp.bfloat16),
                            preferred_element_type=jnp.float32) / l
                outs.append(o)
            attn = jnp.concatenate(outs, axis=1).astype(jnp.bfloat16)

            for k in range(1, N_DEV):
                pr = (my + k) % N_DEV
                chunk = jnp.dot(
                    attn, wo_ref[:, pl.ds(pr * CHW, CHW)].astype(
                        jnp.bfloat16), preferred_element_type=jnp.float32)
                send_buf[rb, k - 1] = chunk.astype(jnp.bfloat16)
                rs = pltpu.make_async_remote_copy(
                    src_ref=send_buf.at[rb, k - 1],
                    dst_ref=rs_buf.at[rb, my],
                    send_sem=send_sems.at[rb, k - 1],
                    recv_sem=rs_sems.at[rb, my],
                    device_id=(pr,),
                    device_id_type=pl.DeviceIdType.MESH,
                )
                rs.start()
                sends.append(rs)

            own[rb] = jnp.dot(
                attn, wo_ref[:, pl.ds(my * CHW, CHW)].astype(jnp.bfloat16),
                preferred_element_type=jnp.float32)
            rs_buf[rb, my] = jnp.zeros((RB, CHW), jnp.bfloat16)

        for rb in range(ROWB):
            r0 = rb * RB
            for k in range(1, N_DEV):
                s = (my + k) % N_DEV
                pltpu.make_async_remote_copy(
                    src_ref=rs_buf.at[rb, s],
                    dst_ref=rs_buf.at[rb, s],
                    send_sem=send_sems.at[rb, k - 1],
                    recv_sem=rs_sems.at[rb, s],
                    device_id=(s,),
                    device_id_type=pl.DeviceIdType.MESH,
                ).wait_recv()

            red = own[rb] + jnp.sum(rs_buf[rb].astype(jnp.float32), axis=0)
            out_ref[0, pl.ds(r0, RB), pl.ds(my * CHW, CHW)] = red
            ag_buf[pl.ds(r0, RB), pl.ds(my * CHW, CHW)] = (
                red.astype(jnp.bfloat16))
            for k in range(1, N_DEV):
                pr = (my + k) % N_DEV
                ag = pltpu.make_async_remote_copy(
                    src_ref=ag_buf.at[pl.ds(r0, RB), pl.ds(my * CHW, CHW)],
                    dst_ref=ag_buf.at[pl.ds(r0, RB), pl.ds(my * CHW, CHW)],
                    send_sem=send_sems.at[rb, N_DEV - 1 + k - 1],
                    recv_sem=ag_sems.at[rb, my],
                    device_id=(pr,),
                    device_id_type=pl.DeviceIdType.MESH,
                )
                ag.start()
                sends.append(ag)

        for rb in range(ROWB):
            r0 = rb * RB
            for k in range(1, N_DEV):
                s = (my + k) % N_DEV
                pltpu.make_async_remote_copy(
                    src_ref=ag_buf.at[pl.ds(r0, RB), pl.ds(s * CHW, CHW)],
                    dst_ref=ag_buf.at[pl.ds(r0, RB), pl.ds(s * CHW, CHW)],
                    send_sem=send_sems.at[rb, k - 1],
                    recv_sem=ag_sems.at[rb, s],
                    device_id=(s,),
                    device_id_type=pl.DeviceIdType.MESH,
                ).wait_recv()
                out_ref[0, pl.ds(r0, RB), pl.ds(s * CHW, CHW)] = (
                    ag_buf[pl.ds(r0, RB), pl.ds(s * CHW, CHW)].astype(
                        jnp.float32))

        for rdma in sends:
            rdma.wait_send()

    return pl.pallas_call(
        body,
        out_shape=jax.ShapeDtypeStruct((1, SQ, D_MODEL), jnp.float32),
        in_specs=[
            pl.BlockSpec(memory_space=pltpu.VMEM),
            pl.BlockSpec(memory_space=pltpu.VMEM),
            pl.BlockSpec(memory_space=pltpu.VMEM),
            pl.BlockSpec(memory_space=pl.ANY),
            pl.BlockSpec(memory_space=pl.ANY),
        ],
        out_specs=pl.BlockSpec(memory_space=pltpu.VMEM),
        scratch_shapes=[
            pltpu.VMEM((H_PER, SKV, DH), jnp.float32),
            pltpu.VMEM((H_PER, SKV, DH), jnp.float32),
            pltpu.VMEM((ROWB, N_DEV - 1, RB, CHW), jnp.bfloat16),
            pltpu.VMEM((ROWB, N_DEV, RB, CHW), jnp.bfloat16),
            pltpu.VMEM((SQ, D_MODEL), jnp.bfloat16),
            pltpu.SemaphoreType.DMA((2 * H_PER,)),
            pltpu.SemaphoreType.DMA((ROWB, 2 * (N_DEV - 1))),
            pltpu.SemaphoreType.DMA((ROWB, N_DEV)),
            pltpu.SemaphoreType.DMA((ROWB, N_DEV)),
        ],
        compiler_params=pltpu.CompilerParams(
            collective_id=0,
            vmem_limit_bytes=60 * 1024 * 1024,
        ),
    )(x, Wq, Wo, K_ext, V_ext)
